# Initial kernel scaffold; baseline (speedup 1.0000x reference)
import jax
import jax.numpy as jnp
from jax.experimental import pallas as pl


def kernel(nw_out):
    raise NotImplementedError("write your pallas kernel here")


# speedup vs baseline: 194.8438x; 194.8438x over previous
import jax
import jax.numpy as jnp
from jax.experimental import pallas as pl
from jax.experimental.pallas import tpu as pltpu

_RATIO = 0.2


def _msiw_body(x_ref, out_ref, cnt_ref, ssum_ref, *, nsteps, c, np_total, n_batch):
    i = pl.program_id(0)

    @pl.when(i == 0)
    def _init():
        cnt_ref[...] = jnp.zeros_like(cnt_ref)
        ssum_ref[...] = jnp.zeros_like(ssum_ref)

    x0 = x_ref[0, 0]
    m = x0
    pred = jnp.zeros(x0.shape, dtype=jnp.int32)
    for ci in range(1, c):
        xc = x_ref[0, ci]
        gt = xc > m
        m = jnp.where(gt, xc, m)
        pred = jnp.where(gt, ci, pred)

    z = jnp.zeros_like(m)
    s2 = jnp.zeros_like(m)
    for ci in range(c):
        e = jnp.exp(x_ref[0, ci] - m)
        z += e
        s2 += e * e
    s = s2 / (z * z)

    for ci in range(c):
        hit = pred == ci
        cnt_ref[ci : ci + 1, :] += jnp.sum(
            hit.astype(jnp.float32), axis=0, keepdims=True
        )
        ssum_ref[ci : ci + 1, :] += jnp.sum(
            jnp.where(hit, s, 0.0), axis=0, keepdims=True
        )

    @pl.when(i == nsteps - 1)
    def _finish():
        cnt_t = jnp.sum(cnt_ref[...], axis=1, keepdims=True)
        s_t = jnp.sum(ssum_ref[...], axis=1, keepdims=True)
        np_pow = float(np_total) ** (1.0 - _RATIO)
        pos = cnt_t > 0.0
        den_raw = jnp.exp(_RATIO * jnp.log(jnp.where(pos, cnt_t, 1.0))) * np_pow
        den = jnp.maximum(jnp.where(pos, den_raw, 0.0), 1.0)
        total = jnp.sum(s_t / den, axis=0, keepdims=True)
        out_ref[...] = -total / (n_batch * c)


def kernel(nw_out):
    n, c, h, w = nw_out.shape
    bh = 64
    nh = h // bh
    nsteps = n * nh
    np_total = n * h * w

    import functools

    body = functools.partial(
        _msiw_body, nsteps=nsteps, c=c, np_total=np_total, n_batch=n
    )
    out = pl.pallas_call(
        body,
        grid=(nsteps,),
        in_specs=[
            pl.BlockSpec((1, c, bh, w), lambda i: (i // nh, 0, i % nh, 0)),
        ],
        out_specs=pl.BlockSpec((1, 1), lambda i: (0, 0)),
        out_shape=jax.ShapeDtypeStruct((1, 1), jnp.float32),
        scratch_shapes=[
            pltpu.VMEM((c, w), jnp.float32),
            pltpu.VMEM((c, w), jnp.float32),
        ],
        compiler_params=pltpu.CompilerParams(
            dimension_semantics=("arbitrary",),
        ),
    )(nw_out)
    return out[0, 0]
